# baseline (speedup 1.0000x reference)
import functools

import jax
import jax.numpy as jnp
from jax import lax
from jax.experimental import pallas as pl
from jax.experimental.pallas import tpu as pltpu
from jax.experimental.pallas import tpu_sc as plsc

_L = 16
_CHUNK = 128


@functools.lru_cache(maxsize=None)
def _make_sc_kernel(batch: int, embed: int):
    mesh = plsc.VectorSubcoreMesh(core_axis_name="c", subcore_axis_name="s")
    nw = mesh.num_cores * mesh.num_subcores
    assert batch % (_CHUNK * nw) == 0 and embed == 32
    bpw = batch // nw
    emb_per_quad = 128 // embed

    def body(ui_hbm, mi_hbm, uer_hbm, mer_hbm, ubr_hbm, mbr_hbm,
             ue_hbm, me_hbm, ub_hbm, mb_hbm, out_hbm,
             ui_v, mi_v, uer_v, mer_v, ubr_v, mbr_v,
             ue_v, me_v, ub_v, mb_v, out_v, sem):
        wid = lax.axis_index("s") * mesh.num_cores + lax.axis_index("c")
        base = wid * bpw
        pltpu.sync_copy(ui_hbm.at[pl.ds(base, bpw)], ui_v)
        pltpu.sync_copy(mi_hbm.at[pl.ds(base, bpw)], mi_v)
        pltpu.sync_copy(uer_hbm.at[pl.ds(base, bpw)], uer_v)
        pltpu.sync_copy(mer_hbm.at[pl.ds(base, bpw)], mer_v)
        pltpu.sync_copy(ubr_hbm.at[pl.ds(base, bpw)], ubr_v)
        pltpu.sync_copy(mbr_hbm.at[pl.ds(base, bpw)], mbr_v)

        lane = lax.iota(jnp.int32, _L)

        def chunk_body(c, carry):
            cb = c * _CHUNK
            c1 = pltpu.async_copy(ue_hbm.at[uer_v.at[pl.ds(cb, _CHUNK)]],
                                  ue_v, sem)
            c2 = pltpu.async_copy(me_hbm.at[mer_v.at[pl.ds(cb, _CHUNK)]],
                                  me_v, sem)
            c3 = pltpu.async_copy(ub_hbm.at[ubr_v.at[pl.ds(cb, _CHUNK)]],
                                  ub_v, sem)
            c4 = pltpu.async_copy(mb_hbm.at[mbr_v.at[pl.ds(cb, _CHUNK)]],
                                  mb_v, sem)
            c1.wait()
            c2.wait()
            c3.wait()
            c4.wait()

            def g_body(g, carry2):
                row = g * _L + lane
                ui = ui_v[pl.ds(cb + g * _L, _L)]
                mi = mi_v[pl.ds(cb + g * _L, _L)]
                uoff = (ui & (emb_per_quad - 1)) * embed
                moff = (mi & (emb_per_quad - 1)) * embed
                acc = (plsc.load_gather(ue_v, [row, uoff])
                       * plsc.load_gather(me_v, [row, moff]))
                for d in range(1, embed):
                    acc = acc + (plsc.load_gather(ue_v, [row, uoff + d])
                                 * plsc.load_gather(me_v, [row, moff + d]))
                ub = plsc.load_gather(ub_v, [row, ui & 127])
                mb = plsc.load_gather(mb_v, [row, mi & 127])
                r = acc + ub + mb
                out_v[pl.ds(cb + g * _L, _L)] = jnp.minimum(
                    jnp.maximum(r, jnp.full((_L,), 0.0, jnp.float32)),
                    jnp.full((_L,), 5.0, jnp.float32))
                return carry2

            lax.fori_loop(0, _CHUNK // _L, g_body, 0)
            return carry

        lax.fori_loop(0, bpw // _CHUNK, chunk_body, 0)
        pltpu.sync_copy(out_v, out_hbm.at[pl.ds(base, bpw)])

    return pl.kernel(
        body,
        out_type=jax.ShapeDtypeStruct((batch,), jnp.float32),
        mesh=mesh,
        compiler_params=pltpu.CompilerParams(
            needs_layout_passes=False, disable_bounds_checks=True),
        scratch_types=[
            pltpu.VMEM((bpw,), jnp.int32),
            pltpu.VMEM((bpw,), jnp.int32),
            pltpu.VMEM((bpw,), jnp.int32),
            pltpu.VMEM((bpw,), jnp.int32),
            pltpu.VMEM((bpw,), jnp.int32),
            pltpu.VMEM((bpw,), jnp.int32),
            pltpu.VMEM((_CHUNK, 128), jnp.float32),
            pltpu.VMEM((_CHUNK, 128), jnp.float32),
            pltpu.VMEM((_CHUNK, 128), jnp.float32),
            pltpu.VMEM((_CHUNK, 128), jnp.float32),
            pltpu.VMEM((bpw,), jnp.float32),
            pltpu.SemaphoreType.DMA,
        ],
    )


def kernel(user_indices, movie_indices, user_emb, movie_emb, user_bias, movie_bias):
    batch = user_indices.shape[0]
    n_users, embed = user_emb.shape
    n_movies = movie_emb.shape[0]
    ui = user_indices.astype(jnp.int32)
    mi = movie_indices.astype(jnp.int32)
    quad = 128 // embed

    def pad128(b):
        flat = b.reshape(-1)
        n = flat.shape[0]
        npad = (-n) % 128
        return jnp.pad(flat, (0, npad)).reshape(-1, 128)

    sc = _make_sc_kernel(batch, embed)
    return sc(ui, mi,
              ui // quad, mi // quad, ui // 128, mi // 128,
              user_emb.reshape(n_users // quad, 128),
              movie_emb.reshape(n_movies // quad, 128),
              pad128(user_bias), pad128(movie_bias))


# speedup vs baseline: 1.0340x; 1.0340x over previous
import functools

import jax
import jax.numpy as jnp
from jax import lax
from jax.experimental import pallas as pl
from jax.experimental.pallas import tpu as pltpu
from jax.experimental.pallas import tpu_sc as plsc

_L = 16


@functools.lru_cache(maxsize=None)
def _make_sc_kernel(batch: int, embed: int):
    mesh = plsc.VectorSubcoreMesh(core_axis_name="c", subcore_axis_name="s")
    nw = mesh.num_cores * mesh.num_subcores
    assert batch % (8 * nw) == 0 and embed % _L == 0
    bpw = batch // nw

    def body(ui_hbm, mi_hbm, ue_hbm, me_hbm, ub_hbm, mb_hbm, out_hbm,
             ui_v, mi_v, ue_v, me_v, ub_v, mb_v, out_v, sem):
        wid = lax.axis_index("s") * mesh.num_cores + lax.axis_index("c")
        base = wid * bpw
        pltpu.sync_copy(ui_hbm.at[pl.ds(base, bpw)], ui_v)
        pltpu.sync_copy(mi_hbm.at[pl.ds(base, bpw)], mi_v)
        c1 = pltpu.async_copy(ue_hbm.at[ui_v], ue_v, sem)
        c2 = pltpu.async_copy(me_hbm.at[mi_v], me_v, sem)
        c3 = pltpu.async_copy(ub_hbm.at[ui_v], ub_v, sem)
        c4 = pltpu.async_copy(mb_hbm.at[mi_v], mb_v, sem)
        c1.wait()
        c2.wait()
        c3.wait()
        c4.wait()

        lane = lax.iota(jnp.int32, _L)

        def g_body(g, carry):
            dots = jnp.full((_L,), 0.0, jnp.float32)
            for j in range(_L):
                i = g * _L + j
                acc = ue_v[i, pl.ds(0, _L)] * me_v[i, pl.ds(0, _L)]
                for h in range(1, embed // _L):
                    acc = acc + (ue_v[i, pl.ds(h * _L, _L)]
                                 * me_v[i, pl.ds(h * _L, _L)])
                dots = jnp.where(lane == j, jnp.sum(acc), dots)
            r = dots + ub_v[pl.ds(g * _L, _L)] + mb_v[pl.ds(g * _L, _L)]
            out_v[pl.ds(g * _L, _L)] = jnp.minimum(
                jnp.maximum(r, jnp.full((_L,), 0.0, jnp.float32)),
                jnp.full((_L,), 5.0, jnp.float32))
            return carry

        lax.fori_loop(0, bpw // _L, g_body, 0)
        pltpu.sync_copy(out_v, out_hbm.at[pl.ds(base, bpw)])

    return pl.kernel(
        body,
        out_type=jax.ShapeDtypeStruct((batch,), jnp.float32),
        mesh=mesh,
        compiler_params=pltpu.CompilerParams(
            needs_layout_passes=False, use_tc_tiling_on_sc=False,
            disable_bounds_checks=True),
        scratch_types=[
            pltpu.VMEM((bpw,), jnp.int32),
            pltpu.VMEM((bpw,), jnp.int32),
            pltpu.VMEM((bpw, embed), jnp.float32),
            pltpu.VMEM((bpw, embed), jnp.float32),
            pltpu.VMEM((bpw,), jnp.float32),
            pltpu.VMEM((bpw,), jnp.float32),
            pltpu.VMEM((bpw,), jnp.float32),
            pltpu.SemaphoreType.DMA,
        ],
    )


def kernel(user_indices, movie_indices, user_emb, movie_emb, user_bias, movie_bias):
    batch = user_indices.shape[0]
    embed = user_emb.shape[1]
    sc = _make_sc_kernel(batch, embed)
    return sc(user_indices.astype(jnp.int32),
              movie_indices.astype(jnp.int32),
              user_emb, movie_emb,
              user_bias.reshape(-1), movie_bias.reshape(-1))
